# baseline (speedup 1.0000x reference)
import jax
import jax.numpy as jnp
from jax import lax
from jax.experimental import pallas as pl
from jax.experimental.pallas import tpu as pltpu
from jax.experimental.pallas import tpu_sc as plsc

N = 10000
NP = 10240
E = 320000
D = 128
G = 16
NC = 2
NS = 16
NW = NC * NS
EPW = E // NW
CH = 80
NCH = EPW // CH
NSP = 10112
SLAB = NSP // NS
BR = 1024
NB = NP // BR
CLM = 2048


def _mm_alpha_body(x_ref, w_ref, asr_ref, adr_ref, h_ref, oas_ref, oad_ref):
    h = jnp.dot(x_ref[...], w_ref[...], preferred_element_type=jnp.float32)
    h_ref[...] = h
    oas_ref[...] = jnp.sum(h * asr_ref[...], axis=1).reshape(1, 1, BR)
    oad_ref[...] = jnp.sum(h * adr_ref[...], axis=1).reshape(1, 1, BR)


def _mm_alpha(x, w, a_s, a_d):
    return pl.pallas_call(
        _mm_alpha_body,
        grid=(NB,),
        in_specs=[pl.BlockSpec((BR, D), lambda i: (i, 0)),
                  pl.BlockSpec((D, D), lambda i: (0, 0)),
                  pl.BlockSpec((1, D), lambda i: (0, 0)),
                  pl.BlockSpec((1, D), lambda i: (0, 0))],
        out_specs=[pl.BlockSpec((BR, D), lambda i: (i, 0)),
                   pl.BlockSpec((1, 1, BR), lambda i: (i, 0, 0)),
                   pl.BlockSpec((1, 1, BR), lambda i: (i, 0, 0))],
        out_shape=[jax.ShapeDtypeStruct((NP, D), jnp.float32),
                   jax.ShapeDtypeStruct((NB, 1, BR), jnp.float32),
                   jax.ShapeDtypeStruct((NB, 1, BR), jnp.float32)],
    )(x, w, a_s, a_d)


DUMP = NSP - 8


def _edge_body(h_hbm, as_hbm, ad_hbm, src_hbm, dst_hbm,
               psum_hbm, pden_hbm,
               as_v, ad_v, den_v, ex_v, rmask_v, rows_v,
               sidx_v, didx_v, ridx_v, claim_v, psum_sh, gsem, ssem):
    cid = lax.axis_index("c")
    sid = lax.axis_index("s")
    wid = cid * NS + sid
    z16 = jnp.zeros((16,), jnp.float32)
    iota16 = lax.iota(jnp.int32, 16)

    pltpu.sync_copy(as_hbm.at[pl.ds(0, NSP)], as_v)
    pltpu.sync_copy(ad_hbm.at[pl.ds(0, NSP)], ad_v)

    def zden(i, c):
        den_v[pl.ds(i * 16, 16)] = z16
        return c
    lax.fori_loop(0, NP // 16, zden, 0)

    def zrows(i, c):
        rows_v[i // (D // 16), pl.ds((i % (D // 16)) * 16, 16)] = z16
        return c
    lax.fori_loop(0, CH * (D // 16), zrows, 0)
    base = sid * SLAB
    for k in range(SLAB // CH):
        pltpu.sync_copy(rows_v, psum_sh.at[pl.ds(base + k * CH, CH), :])
    pltpu.sync_copy(rows_v.at[pl.ds(0, SLAB % CH), :],
                    psum_sh.at[pl.ds(base + (SLAB // CH) * CH, SLAB % CH), :])
    pltpu.sync_copy(rows_v.at[pl.ds(0, (NP - NSP) // NS), :],
                    psum_hbm.at[cid, pl.ds(NSP + sid * ((NP - NSP) // NS),
                                           (NP - NSP) // NS), :])
    plsc.subcore_barrier()

    def p2(c, carry):
        eoff = wid * EPW + c * CH
        pltpu.sync_copy(src_hbm.at[pl.ds(eoff, CH)], sidx_v)
        pltpu.sync_copy(dst_hbm.at[pl.ds(eoff, CH)], didx_v)
        cp = pltpu.async_copy(h_hbm.at[sidx_v], rows_v, gsem)
        anyrem = False
        for j in range(CH // 16):
            s16 = sidx_v[pl.ds(j * 16, 16)]
            d16 = didx_v[pl.ds(j * 16, 16)]
            a = plsc.load_gather(as_v, [s16]) + plsc.load_gather(ad_v, [d16])
            a = jnp.where(a >= 0.0, a, a * jnp.float32(0.2))
            ex = jnp.exp(a)
            ex_v[pl.ds(j * 16, 16)] = ex
            hd = d16 & jnp.int32(CLM - 1)
            plsc.store_scatter(claim_v, [hd], iota16)
            won = plsc.load_gather(claim_v, [hd]) == iota16
            plsc.addupdate_scatter(den_v, [d16], ex, mask=won)
            ridx_v[j, pl.ds(0, 16)] = jnp.where(won, d16, DUMP)
            rem = ~won
            rmask_v[pl.ds(j * 16, 16)] = rem.astype(jnp.int32)
            anyrem = anyrem | jnp.any(rem)
        cp.wait()

        def scale(r, cc):
            exs = plsc.load_gather(ex_v, [jnp.full((16,), r, jnp.int32)])
            for kk in range(D // 16):
                rows_v[r, pl.ds(kk * 16, 16)] = rows_v[r, pl.ds(kk * 16, 16)] * exs
            return cc
        lax.fori_loop(0, CH, scale, 0)

        cps = [pltpu.async_copy(rows_v.at[pl.ds(j * 16, 16), :],
                                psum_sh.at[ridx_v.at[j]], ssem, add=True)
               for j in range(CH // 16)]
        for cpx in cps:
            cpx.wait()

        @pl.when(anyrem)
        def _slow():
            for j in range(CH // 16):
                m0 = rmask_v[pl.ds(j * 16, 16)] != 0

                @pl.when(jnp.any(m0))
                def _grp():
                    d16 = didx_v[pl.ds(j * 16, 16)]
                    ex = ex_v[pl.ds(j * 16, 16)]

                    def wbody(m):
                        hd = d16 & jnp.int32(CLM - 1)
                        plsc.store_scatter(claim_v, [hd], iota16, mask=m)
                        won = (plsc.load_gather(claim_v, [hd]) == iota16) & m
                        ffs = plsc.all_reduce_ffs(m)
                        won = jnp.where(jnp.any(won), won, (iota16 == ffs) & m)
                        plsc.addupdate_scatter(den_v, [d16], ex, mask=won)
                        ridx_v[j, pl.ds(0, 16)] = jnp.where(won, d16, DUMP)
                        pltpu.async_copy(rows_v.at[pl.ds(j * 16, 16), :],
                                         psum_sh.at[ridx_v.at[j]], ssem,
                                         add=True).wait()
                        return m & ~won
                    lax.while_loop(jnp.any, wbody, m0)
        return carry
    lax.fori_loop(0, NCH, p2, 0)

    pltpu.sync_copy(den_v, pden_hbm.at[wid])
    plsc.subcore_barrier()
    pltpu.sync_copy(psum_sh.at[pl.ds(base, SLAB), :],
                    psum_hbm.at[cid, pl.ds(base, SLAB), :])


def _edge(h, asv, adv, src, dst):
    mesh = plsc.VectorSubcoreMesh(core_axis_name="c", subcore_axis_name="s")
    return pl.kernel(
        _edge_body,
        out_type=[jax.ShapeDtypeStruct((NC, NP, D), jnp.float32),
                  jax.ShapeDtypeStruct((NW, NP), jnp.float32)],
        mesh=mesh,
        compiler_params=pltpu.CompilerParams(needs_layout_passes=False),
        scratch_types=[pltpu.VMEM((NSP,), jnp.float32),
                       pltpu.VMEM((NSP,), jnp.float32),
                       pltpu.VMEM((NP,), jnp.float32),
                       pltpu.VMEM((CH,), jnp.float32),
                       pltpu.VMEM((CH,), jnp.int32),
                       pltpu.VMEM((CH, D), jnp.float32),
                       pltpu.VMEM((CH,), jnp.int32),
                       pltpu.VMEM((CH,), jnp.int32),
                       pltpu.VMEM((CH // 16, 16), jnp.int32),
                       pltpu.VMEM((CLM,), jnp.int32),
                       pltpu.VMEM_SHARED((NSP, D), jnp.float32),
                       pltpu.SemaphoreType.DMA,
                       pltpu.SemaphoreType.DMA],
    )(h, asv, adv, src, dst)


def _comb_mm_body(ps_ref, pd_ref, b_ref, w_ref, asr_ref, adr_ref,
                  h_ref, oas_ref, oad_ref):
    p = ps_ref[0] + ps_ref[1]
    den = jnp.sum(pd_ref[...], axis=0)
    x1 = jnp.maximum(p / (den[:, None] + 1e-16) + b_ref[...], 0.0)
    h = jnp.dot(x1, w_ref[...], preferred_element_type=jnp.float32)
    h_ref[...] = h
    oas_ref[...] = jnp.sum(h * asr_ref[...], axis=1).reshape(1, 1, BR)
    oad_ref[...] = jnp.sum(h * adr_ref[...], axis=1).reshape(1, 1, BR)


def _comb_mm(ps, pd, b, w, a_s, a_d):
    return pl.pallas_call(
        _comb_mm_body,
        grid=(NB,),
        in_specs=[pl.BlockSpec((NC, BR, D), lambda i: (0, i, 0)),
                  pl.BlockSpec((NW, BR), lambda i: (0, i)),
                  pl.BlockSpec((1, D), lambda i: (0, 0)),
                  pl.BlockSpec((D, D), lambda i: (0, 0)),
                  pl.BlockSpec((1, D), lambda i: (0, 0)),
                  pl.BlockSpec((1, D), lambda i: (0, 0))],
        out_specs=[pl.BlockSpec((BR, D), lambda i: (i, 0)),
                   pl.BlockSpec((1, 1, BR), lambda i: (i, 0, 0)),
                   pl.BlockSpec((1, 1, BR), lambda i: (i, 0, 0))],
        out_shape=[jax.ShapeDtypeStruct((NP, D), jnp.float32),
                   jax.ShapeDtypeStruct((NB, 1, BR), jnp.float32),
                   jax.ShapeDtypeStruct((NB, 1, BR), jnp.float32)],
    )(ps, pd, b, w, a_s, a_d)


def _pool_body(ps_ref, pd_ref, b_ref, batch_ref, wp1_ref, bp1_ref,
               wp2_ref, bp2_ref, out_ref, acc, cnt):
    i = pl.program_id(0)

    @pl.when(i == 0)
    def _():
        acc[...] = jnp.zeros_like(acc)
        cnt[...] = jnp.zeros_like(cnt)

    p = ps_ref[0] + ps_ref[1]
    den = jnp.sum(pd_ref[...], axis=0)
    h2 = jnp.maximum(p / (den[:, None] + 1e-16) + b_ref[...], 0.0)
    bb = batch_ref[...].reshape(1, BR)
    iot = lax.broadcasted_iota(jnp.int32, (G, BR), 0)
    oh = (iot == bb).astype(jnp.float32)
    acc[...] += lax.dot_general(oh, h2, (((1,), (0,)), ((), ())),
                                preferred_element_type=jnp.float32)
    cnt[...] += jnp.dot(oh, jnp.ones((BR, D), jnp.float32),
                        preferred_element_type=jnp.float32)

    @pl.when(i == pl.num_programs(0) - 1)
    def _():
        pooled = acc[...] / jnp.maximum(cnt[...], 1.0)
        z = jnp.maximum(jnp.dot(pooled, wp1_ref[...],
                                preferred_element_type=jnp.float32)
                        + bp1_ref[...], 0.0)
        out_ref[...] = (jnp.dot(z, wp2_ref[...],
                                preferred_element_type=jnp.float32)
                        + bp2_ref[...])


def _pool(ps, pd, b, batch3, wp1, bp1, wp2p, bp2p):
    return pl.pallas_call(
        _pool_body,
        grid=(NB,),
        in_specs=[pl.BlockSpec((NC, BR, D), lambda i: (0, i, 0)),
                  pl.BlockSpec((NW, BR), lambda i: (0, i)),
                  pl.BlockSpec((1, D), lambda i: (0, 0)),
                  pl.BlockSpec((1, 1, BR), lambda i: (i, 0, 0)),
                  pl.BlockSpec((D, D), lambda i: (0, 0)),
                  pl.BlockSpec((1, D), lambda i: (0, 0)),
                  pl.BlockSpec((D, D), lambda i: (0, 0)),
                  pl.BlockSpec((1, D), lambda i: (0, 0))],
        out_specs=pl.BlockSpec((G, D), lambda i: (0, 0)),
        out_shape=jax.ShapeDtypeStruct((G, D), jnp.float32),
        scratch_shapes=[pltpu.VMEM((G, D), jnp.float32),
                        pltpu.VMEM((G, D), jnp.float32)],
    )(ps, pd, b, batch3, wp1, bp1, wp2p, bp2p)


def kernel(x, edge_index, batch, W0, a_src0, a_dst0, b0,
           W1, a_src1, a_dst1, b1, Wp1, bp1, Wp2, bp2):
    xp = jnp.pad(x, ((0, NP - N), (0, 0)))
    batch3 = jnp.pad(batch, (0, NP - N), constant_values=G).reshape(NB, 1, BR)
    src = edge_index[0]
    dst = edge_index[1]

    h0, as0, ad0 = _mm_alpha(xp, W0, a_src0, a_dst0)
    ps0, pd0 = _edge(h0, as0.reshape(NP), ad0.reshape(NP), src, dst)
    h1, as1, ad1 = _comb_mm(ps0, pd0, b0.reshape(1, D), W1, a_src1, a_dst1)
    ps1, pd1 = _edge(h1, as1.reshape(NP), ad1.reshape(NP), src, dst)

    wp2p = jnp.pad(Wp2, ((0, 0), (0, D - 1)))
    bp2p = jnp.pad(bp2, (0, D - 1)).reshape(1, D)
    out = _pool(ps1, pd1, b1.reshape(1, D), batch3,
                Wp1, bp1.reshape(1, D), wp2p, bp2p)
    return out[:, :1]


# speedup vs baseline: 1.0230x; 1.0230x over previous
import jax
import jax.numpy as jnp
from jax import lax
from jax.experimental import pallas as pl
from jax.experimental.pallas import tpu as pltpu
from jax.experimental.pallas import tpu_sc as plsc

N = 10000
NP = 10240
E = 320000
D = 128
G = 16
NC = 2
NS = 16
NW = NC * NS
EPW = E // NW
CH = 80
NCH = EPW // CH
NSP = 10112
SLAB = NSP // NS
BR = 1024
NB = NP // BR


def _mm_alpha_body(x_ref, w_ref, asr_ref, adr_ref, h_ref, oas_ref, oad_ref):
    h = jnp.dot(x_ref[...], w_ref[...], preferred_element_type=jnp.float32)
    h_ref[...] = h
    oas_ref[...] = jnp.sum(h * asr_ref[...], axis=1).reshape(1, 1, BR)
    oad_ref[...] = jnp.sum(h * adr_ref[...], axis=1).reshape(1, 1, BR)


def _mm_alpha(x, w, a_s, a_d):
    return pl.pallas_call(
        _mm_alpha_body,
        grid=(NB,),
        in_specs=[pl.BlockSpec((BR, D), lambda i: (i, 0)),
                  pl.BlockSpec((D, D), lambda i: (0, 0)),
                  pl.BlockSpec((1, D), lambda i: (0, 0)),
                  pl.BlockSpec((1, D), lambda i: (0, 0))],
        out_specs=[pl.BlockSpec((BR, D), lambda i: (i, 0)),
                   pl.BlockSpec((1, 1, BR), lambda i: (i, 0, 0)),
                   pl.BlockSpec((1, 1, BR), lambda i: (i, 0, 0))],
        out_shape=[jax.ShapeDtypeStruct((NP, D), jnp.float32),
                   jax.ShapeDtypeStruct((NB, 1, BR), jnp.float32),
                   jax.ShapeDtypeStruct((NB, 1, BR), jnp.float32)],
    )(x, w, a_s, a_d)


def _edge_body(h_hbm, as_hbm, ad_hbm, src_hbm, dst_hbm,
               psum_hbm, pden_hbm,
               as_v, ad_v, den_v, ex_v, rows_v,
               sidx_v, didx_v, psum_sh, gsem, ssem):
    cid = lax.axis_index("c")
    sid = lax.axis_index("s")
    wid = cid * NS + sid
    z16 = jnp.zeros((16,), jnp.float32)

    pltpu.sync_copy(as_hbm.at[pl.ds(0, NSP)], as_v)
    pltpu.sync_copy(ad_hbm.at[pl.ds(0, NSP)], ad_v)

    def zden(i, c):
        den_v[pl.ds(i * 16, 16)] = z16
        return c
    lax.fori_loop(0, NP // 16, zden, 0)

    def zrows(i, c):
        rows_v[i // (D // 16), pl.ds((i % (D // 16)) * 16, 16)] = z16
        return c
    lax.fori_loop(0, CH * (D // 16), zrows, 0)
    base = sid * SLAB
    for k in range(SLAB // CH):
        pltpu.sync_copy(rows_v, psum_sh.at[pl.ds(base + k * CH, CH), :])
    pltpu.sync_copy(rows_v.at[pl.ds(0, SLAB % CH), :],
                    psum_sh.at[pl.ds(base + (SLAB // CH) * CH, SLAB % CH), :])
    pltpu.sync_copy(rows_v.at[pl.ds(0, (NP - NSP) // NS), :],
                    psum_hbm.at[cid, pl.ds(NSP + sid * ((NP - NSP) // NS),
                                           (NP - NSP) // NS), :])
    plsc.subcore_barrier()

    def p2(c, carry):
        eoff = wid * EPW + c * CH
        pltpu.sync_copy(src_hbm.at[pl.ds(eoff, CH)], sidx_v)
        pltpu.sync_copy(dst_hbm.at[pl.ds(eoff, CH)], didx_v)
        cp = pltpu.async_copy(h_hbm.at[sidx_v], rows_v, gsem)
        for j in range(CH // 16):
            s16 = sidx_v[pl.ds(j * 16, 16)]
            d16 = didx_v[pl.ds(j * 16, 16)]
            a = plsc.load_gather(as_v, [s16]) + plsc.load_gather(ad_v, [d16])
            a = jnp.where(a >= 0.0, a, a * jnp.float32(0.2))
            ex = jnp.exp(a)
            ex_v[pl.ds(j * 16, 16)] = ex
            plsc.addupdate_scatter(den_v, [d16], ex)
        cp.wait()

        def scale(r, cc):
            exs = plsc.load_gather(ex_v, [jnp.full((16,), r, jnp.int32)])
            for kk in range(D // 16):
                rows_v[r, pl.ds(kk * 16, 16)] = rows_v[r, pl.ds(kk * 16, 16)] * exs
            return cc
        lax.fori_loop(0, CH, scale, 0)

        pltpu.async_copy(rows_v, psum_sh.at[didx_v], ssem, add=True).wait()
        return carry
    lax.fori_loop(0, NCH, p2, 0)

    pltpu.sync_copy(den_v, pden_hbm.at[wid])
    plsc.subcore_barrier()
    pltpu.sync_copy(psum_sh.at[pl.ds(base, SLAB), :],
                    psum_hbm.at[cid, pl.ds(base, SLAB), :])


def _edge(h, asv, adv, src, dst):
    mesh = plsc.VectorSubcoreMesh(core_axis_name="c", subcore_axis_name="s")
    return pl.kernel(
        _edge_body,
        out_type=[jax.ShapeDtypeStruct((NC, NP, D), jnp.float32),
                  jax.ShapeDtypeStruct((NW, NP), jnp.float32)],
        mesh=mesh,
        compiler_params=pltpu.CompilerParams(needs_layout_passes=False),
        scratch_types=[pltpu.VMEM((NSP,), jnp.float32),
                       pltpu.VMEM((NSP,), jnp.float32),
                       pltpu.VMEM((NP,), jnp.float32),
                       pltpu.VMEM((CH,), jnp.float32),
                       pltpu.VMEM((CH, D), jnp.float32),
                       pltpu.VMEM((CH,), jnp.int32),
                       pltpu.VMEM((CH,), jnp.int32),
                       pltpu.VMEM_SHARED((NSP, D), jnp.float32),
                       pltpu.SemaphoreType.DMA,
                       pltpu.SemaphoreType.DMA],
    )(h, asv, adv, src, dst)


def _comb_mm_body(ps_ref, pd_ref, b_ref, w_ref, asr_ref, adr_ref,
                  h_ref, oas_ref, oad_ref):
    p = ps_ref[0] + ps_ref[1]
    den = jnp.sum(pd_ref[...], axis=0)
    x1 = jnp.maximum(p / (den[:, None] + 1e-16) + b_ref[...], 0.0)
    h = jnp.dot(x1, w_ref[...], preferred_element_type=jnp.float32)
    h_ref[...] = h
    oas_ref[...] = jnp.sum(h * asr_ref[...], axis=1).reshape(1, 1, BR)
    oad_ref[...] = jnp.sum(h * adr_ref[...], axis=1).reshape(1, 1, BR)


def _comb_mm(ps, pd, b, w, a_s, a_d):
    return pl.pallas_call(
        _comb_mm_body,
        grid=(NB,),
        in_specs=[pl.BlockSpec((NC, BR, D), lambda i: (0, i, 0)),
                  pl.BlockSpec((NW, BR), lambda i: (0, i)),
                  pl.BlockSpec((1, D), lambda i: (0, 0)),
                  pl.BlockSpec((D, D), lambda i: (0, 0)),
                  pl.BlockSpec((1, D), lambda i: (0, 0)),
                  pl.BlockSpec((1, D), lambda i: (0, 0))],
        out_specs=[pl.BlockSpec((BR, D), lambda i: (i, 0)),
                   pl.BlockSpec((1, 1, BR), lambda i: (i, 0, 0)),
                   pl.BlockSpec((1, 1, BR), lambda i: (i, 0, 0))],
        out_shape=[jax.ShapeDtypeStruct((NP, D), jnp.float32),
                   jax.ShapeDtypeStruct((NB, 1, BR), jnp.float32),
                   jax.ShapeDtypeStruct((NB, 1, BR), jnp.float32)],
    )(ps, pd, b, w, a_s, a_d)


def _pool_body(ps_ref, pd_ref, b_ref, batch_ref, wp1_ref, bp1_ref,
               wp2_ref, bp2_ref, out_ref, acc, cnt):
    i = pl.program_id(0)

    @pl.when(i == 0)
    def _():
        acc[...] = jnp.zeros_like(acc)
        cnt[...] = jnp.zeros_like(cnt)

    p = ps_ref[0] + ps_ref[1]
    den = jnp.sum(pd_ref[...], axis=0)
    h2 = jnp.maximum(p / (den[:, None] + 1e-16) + b_ref[...], 0.0)
    bb = batch_ref[...].reshape(1, BR)
    iot = lax.broadcasted_iota(jnp.int32, (G, BR), 0)
    oh = (iot == bb).astype(jnp.float32)
    acc[...] += lax.dot_general(oh, h2, (((1,), (0,)), ((), ())),
                                preferred_element_type=jnp.float32)
    cnt[...] += jnp.dot(oh, jnp.ones((BR, D), jnp.float32),
                        preferred_element_type=jnp.float32)

    @pl.when(i == pl.num_programs(0) - 1)
    def _():
        pooled = acc[...] / jnp.maximum(cnt[...], 1.0)
        z = jnp.maximum(jnp.dot(pooled, wp1_ref[...],
                                preferred_element_type=jnp.float32)
                        + bp1_ref[...], 0.0)
        out_ref[...] = (jnp.dot(z, wp2_ref[...],
                                preferred_element_type=jnp.float32)
                        + bp2_ref[...])


def _pool(ps, pd, b, batch3, wp1, bp1, wp2p, bp2p):
    return pl.pallas_call(
        _pool_body,
        grid=(NB,),
        in_specs=[pl.BlockSpec((NC, BR, D), lambda i: (0, i, 0)),
                  pl.BlockSpec((NW, BR), lambda i: (0, i)),
                  pl.BlockSpec((1, D), lambda i: (0, 0)),
                  pl.BlockSpec((1, 1, BR), lambda i: (i, 0, 0)),
                  pl.BlockSpec((D, D), lambda i: (0, 0)),
                  pl.BlockSpec((1, D), lambda i: (0, 0)),
                  pl.BlockSpec((D, D), lambda i: (0, 0)),
                  pl.BlockSpec((1, D), lambda i: (0, 0))],
        out_specs=pl.BlockSpec((G, D), lambda i: (0, 0)),
        out_shape=jax.ShapeDtypeStruct((G, D), jnp.float32),
        scratch_shapes=[pltpu.VMEM((G, D), jnp.float32),
                        pltpu.VMEM((G, D), jnp.float32)],
    )(ps, pd, b, batch3, wp1, bp1, wp2p, bp2p)


def kernel(x, edge_index, batch, W0, a_src0, a_dst0, b0,
           W1, a_src1, a_dst1, b1, Wp1, bp1, Wp2, bp2):
    xp = jnp.pad(x, ((0, NP - N), (0, 0)))
    batch3 = jnp.pad(batch, (0, NP - N), constant_values=G).reshape(NB, 1, BR)
    src = edge_index[0]
    dst = edge_index[1]

    h0, as0, ad0 = _mm_alpha(xp, W0, a_src0, a_dst0)
    ps0, pd0 = _edge(h0, as0.reshape(NP), ad0.reshape(NP), src, dst)
    h1, as1, ad1 = _comb_mm(ps0, pd0, b0.reshape(1, D), W1, a_src1, a_dst1)
    ps1, pd1 = _edge(h1, as1.reshape(NP), ad1.reshape(NP), src, dst)

    wp2p = jnp.pad(Wp2, ((0, 0), (0, D - 1)))
    bp2p = jnp.pad(bp2, (0, D - 1)).reshape(1, D)
    out = _pool(ps1, pd1, b1.reshape(1, D), batch3,
                Wp1, bp1.reshape(1, D), wp2p, bp2p)
    return out[:, :1]


# speedup vs baseline: 1.1475x; 1.1218x over previous
import jax
import jax.numpy as jnp
from jax import lax
from jax.experimental import pallas as pl
from jax.experimental.pallas import tpu as pltpu
from jax.experimental.pallas import tpu_sc as plsc

N = 10000
NP = 10240
E = 320000
D = 128
G = 16
NC = 2
NS = 16
NW = NC * NS
EPW = E // NW
CH = 80
NCH = EPW // CH
NSP = 10112
SLAB = NSP // NS
BR = 1024
NB = NP // BR


def _mm_alpha_body(x_ref, w_ref, asr_ref, adr_ref, h_ref, oas_ref, oad_ref):
    h = jnp.dot(x_ref[...], w_ref[...], preferred_element_type=jnp.float32)
    h_ref[...] = h
    oas_ref[...] = jnp.sum(h * asr_ref[...], axis=1).reshape(1, 1, BR)
    oad_ref[...] = jnp.sum(h * adr_ref[...], axis=1).reshape(1, 1, BR)


def _mm_alpha(x, w, a_s, a_d):
    return pl.pallas_call(
        _mm_alpha_body,
        grid=(NB,),
        in_specs=[pl.BlockSpec((BR, D), lambda i: (i, 0)),
                  pl.BlockSpec((D, D), lambda i: (0, 0)),
                  pl.BlockSpec((1, D), lambda i: (0, 0)),
                  pl.BlockSpec((1, D), lambda i: (0, 0))],
        out_specs=[pl.BlockSpec((BR, D), lambda i: (i, 0)),
                   pl.BlockSpec((1, 1, BR), lambda i: (i, 0, 0)),
                   pl.BlockSpec((1, 1, BR), lambda i: (i, 0, 0))],
        out_shape=[jax.ShapeDtypeStruct((NP, D), jnp.float32),
                   jax.ShapeDtypeStruct((NB, 1, BR), jnp.float32),
                   jax.ShapeDtypeStruct((NB, 1, BR), jnp.float32)],
    )(x, w, a_s, a_d)


def _edge_body(h_hbm, as_hbm, ad_hbm, src_hbm, dst_hbm,
               psum_hbm, pden_hbm,
               as_v, ad_v, den_v, ex_v, rows_v,
               sidx_v, didx_v, psum_sh, gsem, ssem):
    cid = lax.axis_index("c")
    sid = lax.axis_index("s")
    wid = cid * NS + sid
    z16 = jnp.zeros((16,), jnp.float32)

    pltpu.sync_copy(as_hbm.at[pl.ds(0, NSP)], as_v)
    pltpu.sync_copy(ad_hbm.at[pl.ds(0, NSP)], ad_v)

    def zden(i, c):
        den_v[pl.ds(i * 16, 16)] = z16
        return c
    lax.fori_loop(0, NP // 16, zden, 0)

    def zrows(i, c):
        rows_v[i // (D // 16), pl.ds((i % (D // 16)) * 16, 16)] = z16
        return c
    lax.fori_loop(0, CH * (D // 16), zrows, 0)
    base = sid * SLAB
    for k in range(SLAB // CH):
        pltpu.sync_copy(rows_v, psum_sh.at[pl.ds(base + k * CH, CH), :])
    pltpu.sync_copy(rows_v.at[pl.ds(0, SLAB % CH), :],
                    psum_sh.at[pl.ds(base + (SLAB // CH) * CH, SLAB % CH), :])
    pltpu.sync_copy(rows_v.at[pl.ds(0, (NP - NSP) // NS), :],
                    psum_hbm.at[cid, pl.ds(NSP + sid * ((NP - NSP) // NS),
                                           (NP - NSP) // NS), :])
    plsc.subcore_barrier()

    def p2(c, carry):
        eoff = wid * EPW + c * CH
        pltpu.sync_copy(src_hbm.at[pl.ds(eoff, CH)], sidx_v)
        pltpu.sync_copy(dst_hbm.at[pl.ds(eoff, CH)], didx_v)
        cp = pltpu.async_copy(h_hbm.at[sidx_v], rows_v, gsem)
        for j in range(CH // 16):
            s16 = sidx_v[pl.ds(j * 16, 16)]
            d16 = didx_v[pl.ds(j * 16, 16)]
            a = plsc.load_gather(as_v, [s16]) + plsc.load_gather(ad_v, [d16])
            a = jnp.where(a >= 0.0, a, a * jnp.float32(0.2))
            ex = jnp.exp(a)
            ex_v[pl.ds(j * 16, 16)] = ex
            plsc.addupdate_scatter(den_v, [d16], ex)
        cp.wait()

        @plsc.parallel_loop(0, CH, unroll=8)
        def _scale(r):
            exs = plsc.load_gather(ex_v, [jnp.full((16,), r, jnp.int32)])
            for kk in range(D // 16):
                rows_v[r, pl.ds(kk * 16, 16)] = rows_v[r, pl.ds(kk * 16, 16)] * exs

        pltpu.async_copy(rows_v, psum_sh.at[didx_v], ssem, add=True).wait()
        return carry
    lax.fori_loop(0, NCH, p2, 0)

    pltpu.sync_copy(den_v, pden_hbm.at[wid])
    plsc.subcore_barrier()
    pltpu.sync_copy(psum_sh.at[pl.ds(base, SLAB), :],
                    psum_hbm.at[cid, pl.ds(base, SLAB), :])


def _edge(h, asv, adv, src, dst):
    mesh = plsc.VectorSubcoreMesh(core_axis_name="c", subcore_axis_name="s")
    return pl.kernel(
        _edge_body,
        out_type=[jax.ShapeDtypeStruct((NC, NP, D), jnp.float32),
                  jax.ShapeDtypeStruct((NW, NP), jnp.float32)],
        mesh=mesh,
        compiler_params=pltpu.CompilerParams(needs_layout_passes=False),
        scratch_types=[pltpu.VMEM((NSP,), jnp.float32),
                       pltpu.VMEM((NSP,), jnp.float32),
                       pltpu.VMEM((NP,), jnp.float32),
                       pltpu.VMEM((CH,), jnp.float32),
                       pltpu.VMEM((CH, D), jnp.float32),
                       pltpu.VMEM((CH,), jnp.int32),
                       pltpu.VMEM((CH,), jnp.int32),
                       pltpu.VMEM_SHARED((NSP, D), jnp.float32),
                       pltpu.SemaphoreType.DMA,
                       pltpu.SemaphoreType.DMA],
    )(h, asv, adv, src, dst)


def _comb_mm_body(ps_ref, pd_ref, b_ref, w_ref, asr_ref, adr_ref,
                  h_ref, oas_ref, oad_ref):
    p = ps_ref[0] + ps_ref[1]
    den = jnp.sum(pd_ref[...], axis=0)
    x1 = jnp.maximum(p / (den[:, None] + 1e-16) + b_ref[...], 0.0)
    h = jnp.dot(x1, w_ref[...], preferred_element_type=jnp.float32)
    h_ref[...] = h
    oas_ref[...] = jnp.sum(h * asr_ref[...], axis=1).reshape(1, 1, BR)
    oad_ref[...] = jnp.sum(h * adr_ref[...], axis=1).reshape(1, 1, BR)


def _comb_mm(ps, pd, b, w, a_s, a_d):
    return pl.pallas_call(
        _comb_mm_body,
        grid=(NB,),
        in_specs=[pl.BlockSpec((NC, BR, D), lambda i: (0, i, 0)),
                  pl.BlockSpec((NW, BR), lambda i: (0, i)),
                  pl.BlockSpec((1, D), lambda i: (0, 0)),
                  pl.BlockSpec((D, D), lambda i: (0, 0)),
                  pl.BlockSpec((1, D), lambda i: (0, 0)),
                  pl.BlockSpec((1, D), lambda i: (0, 0))],
        out_specs=[pl.BlockSpec((BR, D), lambda i: (i, 0)),
                   pl.BlockSpec((1, 1, BR), lambda i: (i, 0, 0)),
                   pl.BlockSpec((1, 1, BR), lambda i: (i, 0, 0))],
        out_shape=[jax.ShapeDtypeStruct((NP, D), jnp.float32),
                   jax.ShapeDtypeStruct((NB, 1, BR), jnp.float32),
                   jax.ShapeDtypeStruct((NB, 1, BR), jnp.float32)],
    )(ps, pd, b, w, a_s, a_d)


def _pool_body(ps_ref, pd_ref, b_ref, batch_ref, wp1_ref, bp1_ref,
               wp2_ref, bp2_ref, out_ref, acc, cnt):
    i = pl.program_id(0)

    @pl.when(i == 0)
    def _():
        acc[...] = jnp.zeros_like(acc)
        cnt[...] = jnp.zeros_like(cnt)

    p = ps_ref[0] + ps_ref[1]
    den = jnp.sum(pd_ref[...], axis=0)
    h2 = jnp.maximum(p / (den[:, None] + 1e-16) + b_ref[...], 0.0)
    bb = batch_ref[...].reshape(1, BR)
    iot = lax.broadcasted_iota(jnp.int32, (G, BR), 0)
    oh = (iot == bb).astype(jnp.float32)
    acc[...] += lax.dot_general(oh, h2, (((1,), (0,)), ((), ())),
                                preferred_element_type=jnp.float32)
    cnt[...] += jnp.dot(oh, jnp.ones((BR, D), jnp.float32),
                        preferred_element_type=jnp.float32)

    @pl.when(i == pl.num_programs(0) - 1)
    def _():
        pooled = acc[...] / jnp.maximum(cnt[...], 1.0)
        z = jnp.maximum(jnp.dot(pooled, wp1_ref[...],
                                preferred_element_type=jnp.float32)
                        + bp1_ref[...], 0.0)
        out_ref[...] = (jnp.dot(z, wp2_ref[...],
                                preferred_element_type=jnp.float32)
                        + bp2_ref[...])


def _pool(ps, pd, b, batch3, wp1, bp1, wp2p, bp2p):
    return pl.pallas_call(
        _pool_body,
        grid=(NB,),
        in_specs=[pl.BlockSpec((NC, BR, D), lambda i: (0, i, 0)),
                  pl.BlockSpec((NW, BR), lambda i: (0, i)),
                  pl.BlockSpec((1, D), lambda i: (0, 0)),
                  pl.BlockSpec((1, 1, BR), lambda i: (i, 0, 0)),
                  pl.BlockSpec((D, D), lambda i: (0, 0)),
                  pl.BlockSpec((1, D), lambda i: (0, 0)),
                  pl.BlockSpec((D, D), lambda i: (0, 0)),
                  pl.BlockSpec((1, D), lambda i: (0, 0))],
        out_specs=pl.BlockSpec((G, D), lambda i: (0, 0)),
        out_shape=jax.ShapeDtypeStruct((G, D), jnp.float32),
        scratch_shapes=[pltpu.VMEM((G, D), jnp.float32),
                        pltpu.VMEM((G, D), jnp.float32)],
    )(ps, pd, b, batch3, wp1, bp1, wp2p, bp2p)


def kernel(x, edge_index, batch, W0, a_src0, a_dst0, b0,
           W1, a_src1, a_dst1, b1, Wp1, bp1, Wp2, bp2):
    xp = jnp.pad(x, ((0, NP - N), (0, 0)))
    batch3 = jnp.pad(batch, (0, NP - N), constant_values=G).reshape(NB, 1, BR)
    src = edge_index[0]
    dst = edge_index[1]

    h0, as0, ad0 = _mm_alpha(xp, W0, a_src0, a_dst0)
    ps0, pd0 = _edge(h0, as0.reshape(NP), ad0.reshape(NP), src, dst)
    h1, as1, ad1 = _comb_mm(ps0, pd0, b0.reshape(1, D), W1, a_src1, a_dst1)
    ps1, pd1 = _edge(h1, as1.reshape(NP), ad1.reshape(NP), src, dst)

    wp2p = jnp.pad(Wp2, ((0, 0), (0, D - 1)))
    bp2p = jnp.pad(bp2, (0, D - 1)).reshape(1, D)
    out = _pool(ps1, pd1, b1.reshape(1, D), batch3,
                Wp1, bp1.reshape(1, D), wp2p, bp2p)
    return out[:, :1]


# speedup vs baseline: 1.5198x; 1.3244x over previous
import jax
import jax.numpy as jnp
from jax import lax
from jax.experimental import pallas as pl
from jax.experimental.pallas import tpu as pltpu
from jax.experimental.pallas import tpu_sc as plsc

N = 10000
NP = 10240
E = 320000
D = 128
G = 16
NC = 2
NS = 16
NW = NC * NS
EPW = E // NW
CH = 80
NCH = EPW // CH
HCH = CH // 2
BLK = 25
NBLK = NCH // BLK
NSP = 10112
SLAB = NSP // NS
BR = 1024
NB = NP // BR


def _mm_alpha_body(x_ref, w_ref, asr_ref, adr_ref, h_ref, oas_ref, oad_ref):
    h = jnp.dot(x_ref[...], w_ref[...], preferred_element_type=jnp.float32)
    h_ref[...] = h
    oas_ref[...] = jnp.sum(h * asr_ref[...], axis=1).reshape(1, 1, BR)
    oad_ref[...] = jnp.sum(h * adr_ref[...], axis=1).reshape(1, 1, BR)


def _mm_alpha(x, w, a_s, a_d):
    return pl.pallas_call(
        _mm_alpha_body,
        grid=(NB,),
        in_specs=[pl.BlockSpec((BR, D), lambda i: (i, 0)),
                  pl.BlockSpec((D, D), lambda i: (0, 0)),
                  pl.BlockSpec((1, D), lambda i: (0, 0)),
                  pl.BlockSpec((1, D), lambda i: (0, 0))],
        out_specs=[pl.BlockSpec((BR, D), lambda i: (i, 0)),
                   pl.BlockSpec((1, 1, BR), lambda i: (i, 0, 0)),
                   pl.BlockSpec((1, 1, BR), lambda i: (i, 0, 0))],
        out_shape=[jax.ShapeDtypeStruct((NP, D), jnp.float32),
                   jax.ShapeDtypeStruct((NB, 1, BR), jnp.float32),
                   jax.ShapeDtypeStruct((NB, 1, BR), jnp.float32)],
    )(x, w, a_s, a_d)


def _edge_body(h_hbm, as_hbm, ad_hbm, src_hbm, dst_hbm,
               psum_hbm, pden_hbm,
               as_v, ad_v, den_v, ex_v, rows_a, rows_b,
               sblk, dblk, psum_sh, gsA, gsB, ssA, ssB):
    cid = lax.axis_index("c")
    sid = lax.axis_index("s")
    wid = cid * NS + sid
    z16 = jnp.zeros((16,), jnp.float32)

    pltpu.sync_copy(as_hbm.at[pl.ds(0, NSP)], as_v)
    pltpu.sync_copy(ad_hbm.at[pl.ds(0, NSP)], ad_v)

    def zden(i, c):
        den_v[pl.ds(i * 16, 16)] = z16
        return c
    lax.fori_loop(0, NP // 16, zden, 0)

    @plsc.parallel_loop(0, HCH * (D // 16), unroll=8)
    def _zrows(i):
        rows_a[i // (D // 16), pl.ds((i % (D // 16)) * 16, 16)] = z16
    base = sid * SLAB
    for k in range(SLAB // HCH):
        pltpu.sync_copy(rows_a, psum_sh.at[pl.ds(base + k * HCH, HCH), :])
    pltpu.sync_copy(rows_a.at[pl.ds(0, SLAB % HCH), :],
                    psum_sh.at[pl.ds(base + (SLAB // HCH) * HCH, SLAB % HCH), :])
    pltpu.sync_copy(rows_a.at[pl.ds(0, (NP - NSP) // NS), :],
                    psum_hbm.at[cid, pl.ds(NSP + sid * ((NP - NSP) // NS),
                                           (NP - NSP) // NS), :])
    plsc.subcore_barrier()

    def exden(off):
        for j in range(CH // 16):
            s16 = sblk[pl.ds(off + j * 16, 16)]
            d16 = dblk[pl.ds(off + j * 16, 16)]
            a = plsc.load_gather(as_v, [s16]) + plsc.load_gather(ad_v, [d16])
            a = jnp.where(a >= 0.0, a, a * jnp.float32(0.2))
            ex = jnp.exp(a)
            ex_v[pl.ds(j * 16, 16)] = ex
            plsc.addupdate_scatter(den_v, [d16], ex)

    def scale(rbuf, ebase):
        @plsc.parallel_loop(0, HCH, unroll=8)
        def _s(r):
            exs = plsc.load_gather(ex_v, [jnp.full((16,), r, jnp.int32) + ebase])
            for kk in range(D // 16):
                rbuf[r, pl.ds(kk * 16, 16)] = rbuf[r, pl.ds(kk * 16, 16)] * exs

    def gthr(woff, rbuf, sem):
        return pltpu.async_copy(h_hbm.at[sblk.at[pl.ds(woff, HCH)]], rbuf, sem)

    def gthr_wait(woff, rbuf, sem):
        pltpu.make_async_copy(h_hbm.at[sblk.at[pl.ds(woff, HCH)]], rbuf, sem).wait()

    def sctr(rbuf, woff, sem):
        return pltpu.async_copy(rbuf, psum_sh.at[dblk.at[pl.ds(woff, HCH)]],
                                sem, add=True)

    def sctr_wait(rbuf, woff, sem):
        pltpu.make_async_copy(rbuf, psum_sh.at[dblk.at[pl.ds(woff, HCH)]],
                              sem).wait()

    def block(b, bcarry):
        eoffb = wid * EPW + b * (BLK * CH)
        pltpu.sync_copy(src_hbm.at[pl.ds(eoffb, BLK * CH)], sblk)
        pltpu.sync_copy(dst_hbm.at[pl.ds(eoffb, BLK * CH)], dblk)

        cpA = gthr(0, rows_a, gsA)
        exden(0)
        cpA.wait()
        cpB = gthr(HCH, rows_b, gsB)
        scale(rows_a, 0)
        cpSA = sctr(rows_a, 0, ssA)
        cpB.wait()
        scale(rows_b, HCH)
        sctr(rows_b, HCH, ssB)
        cpSA.wait()
        gthr(CH, rows_a, gsA)

        def chunk(cc, carry):
            off = cc * CH
            exden(off)
            gthr_wait(off, rows_a, gsA)
            sctr_wait(rows_b, off - HCH, ssB)
            cpB2 = gthr(off + HCH, rows_b, gsB)
            scale(rows_a, 0)
            cpSA2 = sctr(rows_a, off, ssA)
            cpB2.wait()
            scale(rows_b, HCH)
            sctr(rows_b, off + HCH, ssB)
            cpSA2.wait()
            gthr(off + CH, rows_a, gsA)
            return carry
        lax.fori_loop(1, BLK - 1, chunk, 0)

        off = (BLK - 1) * CH
        exden(off)
        gthr_wait(off, rows_a, gsA)
        sctr_wait(rows_b, off - HCH, ssB)
        cpB3 = gthr(off + HCH, rows_b, gsB)
        scale(rows_a, 0)
        cpSA3 = sctr(rows_a, off, ssA)
        cpB3.wait()
        scale(rows_b, HCH)
        cpSB3 = sctr(rows_b, off + HCH, ssB)
        cpSA3.wait()
        cpSB3.wait()
        return bcarry
    lax.fori_loop(0, NBLK, block, 0)

    pltpu.sync_copy(den_v, pden_hbm.at[wid])
    plsc.subcore_barrier()
    pltpu.sync_copy(psum_sh.at[pl.ds(base, SLAB), :],
                    psum_hbm.at[cid, pl.ds(base, SLAB), :])


def _edge(h, asv, adv, src, dst):
    mesh = plsc.VectorSubcoreMesh(core_axis_name="c", subcore_axis_name="s")
    return pl.kernel(
        _edge_body,
        out_type=[jax.ShapeDtypeStruct((NC, NP, D), jnp.float32),
                  jax.ShapeDtypeStruct((NW, NP), jnp.float32)],
        mesh=mesh,
        compiler_params=pltpu.CompilerParams(needs_layout_passes=False),
        scratch_types=[pltpu.VMEM((NSP,), jnp.float32),
                       pltpu.VMEM((NSP,), jnp.float32),
                       pltpu.VMEM((NP,), jnp.float32),
                       pltpu.VMEM((CH,), jnp.float32),
                       pltpu.VMEM((HCH, D), jnp.float32),
                       pltpu.VMEM((HCH, D), jnp.float32),
                       pltpu.VMEM((BLK * CH,), jnp.int32),
                       pltpu.VMEM((BLK * CH,), jnp.int32),
                       pltpu.VMEM_SHARED((NSP, D), jnp.float32),
                       pltpu.SemaphoreType.DMA,
                       pltpu.SemaphoreType.DMA,
                       pltpu.SemaphoreType.DMA,
                       pltpu.SemaphoreType.DMA],
    )(h, asv, adv, src, dst)


def _comb_mm_body(ps_ref, pd_ref, b_ref, w_ref, asr_ref, adr_ref,
                  h_ref, oas_ref, oad_ref):
    p = ps_ref[0] + ps_ref[1]
    den = jnp.sum(pd_ref[...], axis=0)
    x1 = jnp.maximum(p / (den[:, None] + 1e-16) + b_ref[...], 0.0)
    h = jnp.dot(x1, w_ref[...], preferred_element_type=jnp.float32)
    h_ref[...] = h
    oas_ref[...] = jnp.sum(h * asr_ref[...], axis=1).reshape(1, 1, BR)
    oad_ref[...] = jnp.sum(h * adr_ref[...], axis=1).reshape(1, 1, BR)


def _comb_mm(ps, pd, b, w, a_s, a_d):
    return pl.pallas_call(
        _comb_mm_body,
        grid=(NB,),
        in_specs=[pl.BlockSpec((NC, BR, D), lambda i: (0, i, 0)),
                  pl.BlockSpec((NW, BR), lambda i: (0, i)),
                  pl.BlockSpec((1, D), lambda i: (0, 0)),
                  pl.BlockSpec((D, D), lambda i: (0, 0)),
                  pl.BlockSpec((1, D), lambda i: (0, 0)),
                  pl.BlockSpec((1, D), lambda i: (0, 0))],
        out_specs=[pl.BlockSpec((BR, D), lambda i: (i, 0)),
                   pl.BlockSpec((1, 1, BR), lambda i: (i, 0, 0)),
                   pl.BlockSpec((1, 1, BR), lambda i: (i, 0, 0))],
        out_shape=[jax.ShapeDtypeStruct((NP, D), jnp.float32),
                   jax.ShapeDtypeStruct((NB, 1, BR), jnp.float32),
                   jax.ShapeDtypeStruct((NB, 1, BR), jnp.float32)],
    )(ps, pd, b, w, a_s, a_d)


def _pool_body(ps_ref, pd_ref, b_ref, batch_ref, wp1_ref, bp1_ref,
               wp2_ref, bp2_ref, out_ref, acc, cnt):
    i = pl.program_id(0)

    @pl.when(i == 0)
    def _():
        acc[...] = jnp.zeros_like(acc)
        cnt[...] = jnp.zeros_like(cnt)

    p = ps_ref[0] + ps_ref[1]
    den = jnp.sum(pd_ref[...], axis=0)
    h2 = jnp.maximum(p / (den[:, None] + 1e-16) + b_ref[...], 0.0)
    bb = batch_ref[...].reshape(1, BR)
    iot = lax.broadcasted_iota(jnp.int32, (G, BR), 0)
    oh = (iot == bb).astype(jnp.float32)
    acc[...] += lax.dot_general(oh, h2, (((1,), (0,)), ((), ())),
                                preferred_element_type=jnp.float32)
    cnt[...] += jnp.dot(oh, jnp.ones((BR, D), jnp.float32),
                        preferred_element_type=jnp.float32)

    @pl.when(i == pl.num_programs(0) - 1)
    def _():
        pooled = acc[...] / jnp.maximum(cnt[...], 1.0)
        z = jnp.maximum(jnp.dot(pooled, wp1_ref[...],
                                preferred_element_type=jnp.float32)
                        + bp1_ref[...], 0.0)
        out_ref[...] = (jnp.dot(z, wp2_ref[...],
                                preferred_element_type=jnp.float32)
                        + bp2_ref[...])


def _pool(ps, pd, b, batch3, wp1, bp1, wp2p, bp2p):
    return pl.pallas_call(
        _pool_body,
        grid=(NB,),
        in_specs=[pl.BlockSpec((NC, BR, D), lambda i: (0, i, 0)),
                  pl.BlockSpec((NW, BR), lambda i: (0, i)),
                  pl.BlockSpec((1, D), lambda i: (0, 0)),
                  pl.BlockSpec((1, 1, BR), lambda i: (i, 0, 0)),
                  pl.BlockSpec((D, D), lambda i: (0, 0)),
                  pl.BlockSpec((1, D), lambda i: (0, 0)),
                  pl.BlockSpec((D, D), lambda i: (0, 0)),
                  pl.BlockSpec((1, D), lambda i: (0, 0))],
        out_specs=pl.BlockSpec((G, D), lambda i: (0, 0)),
        out_shape=jax.ShapeDtypeStruct((G, D), jnp.float32),
        scratch_shapes=[pltpu.VMEM((G, D), jnp.float32),
                        pltpu.VMEM((G, D), jnp.float32)],
    )(ps, pd, b, batch3, wp1, bp1, wp2p, bp2p)


def kernel(x, edge_index, batch, W0, a_src0, a_dst0, b0,
           W1, a_src1, a_dst1, b1, Wp1, bp1, Wp2, bp2):
    xp = jnp.pad(x, ((0, NP - N), (0, 0)))
    batch3 = jnp.pad(batch, (0, NP - N), constant_values=G).reshape(NB, 1, BR)
    src = edge_index[0]
    dst = edge_index[1]

    h0, as0, ad0 = _mm_alpha(xp, W0, a_src0, a_dst0)
    ps0, pd0 = _edge(h0, as0.reshape(NP), ad0.reshape(NP), src, dst)
    h1, as1, ad1 = _comb_mm(ps0, pd0, b0.reshape(1, D), W1, a_src1, a_dst1)
    ps1, pd1 = _edge(h1, as1.reshape(NP), ad1.reshape(NP), src, dst)

    wp2p = jnp.pad(Wp2, ((0, 0), (0, D - 1)))
    bp2p = jnp.pad(bp2, (0, D - 1)).reshape(1, D)
    out = _pool(ps1, pd1, b1.reshape(1, D), batch3,
                Wp1, bp1.reshape(1, D), wp2p, bp2p)
    return out[:, :1]
